# baseline (speedup 1.0000x reference)
import functools

import jax
import jax.numpy as jnp
from jax import lax
from jax.experimental import pallas as pl
from jax.experimental.pallas import tpu as pltpu
from jax.experimental.pallas import tpu_sc as plsc

B = 512
N = 50000
D = 768
K = 5
NB = 3072
NSTEP = (N + NB - 1) // NB
NEG = float("-inf")
_DN = (((1,), (1,)), ((), ()))


CW = 128


def _topk_body(cf_ref, tf_ref, spk_ref, tgt_ref, out_ref,
               qn_ref, cv_ref, ci_ref, io_ref):
    j = pl.program_id(0)

    @pl.when(j == 0)
    def _init():
        cf = cf_ref[...]
        nrm = jnp.sqrt(jnp.sum(cf * cf, axis=1, keepdims=True))
        qn_ref[...] = (cf / jnp.maximum(nrm, 1e-8)).astype(jnp.bfloat16)
        cv_ref[...] = jnp.full((B, CW), NEG, jnp.float32)
        ci_ref[...] = jnp.zeros((B, CW), jnp.float32)
        io_ref[...] = lax.broadcasted_iota(
            jnp.int32, (B, NB), 1).astype(jnp.float32)

    tf = tf_ref[...]
    nrm = jnp.sqrt(jnp.sum(tf * tf, axis=1, keepdims=True))
    cn = (tf / jnp.maximum(nrm, 1e-8)).astype(jnp.bfloat16)
    S = lax.dot_general(qn_ref[...], cn, _DN,
                        preferred_element_type=jnp.float32)
    colid = lax.broadcasted_iota(jnp.int32, (1, NB), 1)
    spk = jnp.where(colid < N - j * NB, spk_ref[...], -1)
    valid = spk == tgt_ref[...]

    offf = (j * NB).astype(jnp.float32)
    St = jnp.concatenate([cv_ref[...], jnp.where(valid, S, NEG)], axis=1)
    It = jnp.concatenate([ci_ref[...], io_ref[...] + offf], axis=1)
    vals, idxs = [], []
    for t in range(K):
        m = jnp.max(St, axis=1, keepdims=True)
        matched = St >= m
        idxf = jnp.min(jnp.where(matched, It, jnp.float32(1e9)),
                       axis=1, keepdims=True)
        vals.append(m)
        idxs.append(idxf)
        if t < K - 1:
            St = jnp.where(matched, NEG, St)

    cv_ref[:, :8] = jnp.concatenate(
        vals + [jnp.full((B, 3), NEG, jnp.float32)], axis=1)
    ixs8 = jnp.concatenate(idxs + [jnp.zeros((B, 3), jnp.float32)], axis=1)
    ci_ref[:, :8] = ixs8

    @pl.when(j == pl.num_programs(0) - 1)
    def _fin():
        out_ref[...] = ixs8.astype(jnp.int32)


def _run_topk(cf, tf, spk2, tgt2):
    return pl.pallas_call(
        _topk_body,
        grid=(NSTEP,),
        in_specs=[
            pl.BlockSpec((B, D), lambda j: (0, 0)),
            pl.BlockSpec((NB, D), lambda j: (j, 0)),
            pl.BlockSpec((1, NB), lambda j: (0, j)),
            pl.BlockSpec((B, 1), lambda j: (0, 0)),
        ],
        out_specs=pl.BlockSpec((B, 8), lambda j: (0, 0)),
        out_shape=jax.ShapeDtypeStruct((B, 8), jnp.int32),
        scratch_shapes=[
            pltpu.VMEM((B, D), jnp.bfloat16),
            pltpu.VMEM((B, CW), jnp.float32),
            pltpu.VMEM((B, CW), jnp.float32),
            pltpu.VMEM((B, NB), jnp.float32),
        ],
        compiler_params=pltpu.CompilerParams(
            dimension_semantics=("arbitrary",)),
    )(cf, tf, spk2, tgt2)


_NC = 2
_NS = 16
_NW = _NC * _NS
_QPW = B // _NW
_RPW = _QPW * 8


def _gather_mean_body(idx_hbm, tab_hbm, out_hbm, idx_v, rows_v, out_v, sem):
    wid = lax.axis_index("s") * _NC + lax.axis_index("c")
    fbase = wid * _RPW
    qbase = wid * _QPW
    pltpu.sync_copy(idx_hbm.at[pl.ds(fbase, _RPW)], idx_v)
    pltpu.async_copy(tab_hbm.at[idx_v], rows_v, sem).wait()

    def qloop(q, _):
        def cloop(c, _):
            col = c * 16
            acc = rows_v[q * 8, pl.ds(col, 16)]
            for k in range(1, K):
                acc = acc + rows_v[q * 8 + k, pl.ds(col, 16)]
            out_v[q, pl.ds(col, 16)] = acc * (1.0 / K)
            return 0
        lax.fori_loop(0, D // 16, cloop, 0)
        return 0
    lax.fori_loop(0, _QPW, qloop, 0)
    pltpu.sync_copy(out_v, out_hbm.at[pl.ds(qbase, _QPW)])


def _run_gather_mean(idx_flat, tf):
    fn = functools.partial(
        pl.kernel,
        mesh=plsc.VectorSubcoreMesh(core_axis_name="c", subcore_axis_name="s"),
        out_type=jax.ShapeDtypeStruct((B, D), jnp.float32),
        scratch_types=[
            pltpu.VMEM((_RPW,), jnp.int32),
            pltpu.VMEM((_RPW, D), jnp.float32),
            pltpu.VMEM((_QPW, D), jnp.float32),
            pltpu.SemaphoreType.DMA,
        ],
    )(_gather_mean_body)
    return fn(idx_flat, tf)


def _mlp_body(cf_ref, rm_ref, w1_ref, b1_ref, w2_ref, b2_ref, out_ref,
              comb_ref):
    comb_ref[:, :D] = cf_ref[...]
    comb_ref[:, D:] = rm_ref[...]
    h = lax.dot_general(comb_ref[...], w1_ref[...], _DN,
                        preferred_element_type=jnp.float32) + b1_ref[...]
    h = h * jax.nn.sigmoid(h)
    out_ref[...] = lax.dot_general(h, w2_ref[...], _DN,
                                   preferred_element_type=jnp.float32) \
        + b2_ref[...]


def _run_mlp(cf, rm, w1, b1, w2, b2):
    return pl.pallas_call(
        _mlp_body,
        out_shape=jax.ShapeDtypeStruct((B, D), jnp.float32),
        scratch_shapes=[pltpu.VMEM((B, 2 * D), jnp.float32)],
    )(cf, rm, w1, b1, w2, b2)


def kernel(content_features, target_speaker_id, training_features,
           speaker_ids, W1, b1, W2, b2):
    cf = content_features.astype(jnp.float32)
    tf = training_features.astype(jnp.float32)
    spk2 = speaker_ids.astype(jnp.int32).reshape(1, N)
    tgt2 = target_speaker_id.astype(jnp.int32).reshape(B, 1)

    top8 = _run_topk(cf, tf, spk2, tgt2)
    idx_flat = top8.reshape(-1)
    rm = _run_gather_mean(idx_flat, tf)

    return _run_mlp(cf, rm, W1, b1.reshape(1, D), W2, b2.reshape(1, D))


# speedup vs baseline: 1.0114x; 1.0114x over previous
import functools

import jax
import jax.numpy as jnp
from jax import lax
from jax.experimental import pallas as pl
from jax.experimental.pallas import tpu as pltpu
from jax.experimental.pallas import tpu_sc as plsc

B = 512
N = 50000
D = 768
K = 5
NB = 2048
NSTEP = (N + NB - 1) // NB
NEG = float("-inf")
_DN = (((1,), (1,)), ((), ()))


CW = 128


def _topk_body(cf_ref, tf_ref, spk_ref, tgt_ref, out_ref,
               qn_ref, cv_ref, ci_ref, io_ref):
    j = pl.program_id(0)

    @pl.when(j == 0)
    def _init():
        cf = cf_ref[...]
        nrm = jnp.sqrt(jnp.sum(cf * cf, axis=1, keepdims=True))
        qn_ref[...] = (cf / jnp.maximum(nrm, 1e-8)).astype(jnp.bfloat16)
        cv_ref[...] = jnp.full((B, CW), NEG, jnp.float32)
        ci_ref[...] = jnp.zeros((B, CW), jnp.float32)
        io_ref[...] = lax.broadcasted_iota(
            jnp.int32, (B, NB), 1).astype(jnp.float32)

    tf = tf_ref[...]
    nrm = jnp.sqrt(jnp.sum(tf * tf, axis=1, keepdims=True))
    cn = (tf / jnp.maximum(nrm, 1e-8)).astype(jnp.bfloat16)
    S = lax.dot_general(qn_ref[...], cn, _DN,
                        preferred_element_type=jnp.float32)
    colid = lax.broadcasted_iota(jnp.int32, (1, NB), 1)
    spk = jnp.where(colid < N - j * NB, spk_ref[...], -1)
    valid = spk == tgt_ref[...]

    offf = (j * NB).astype(jnp.float32)
    St = jnp.concatenate([cv_ref[...], jnp.where(valid, S, NEG)], axis=1)
    It = jnp.concatenate([ci_ref[...], io_ref[...] + offf], axis=1)
    vals, idxs = [], []
    for t in range(K):
        m = jnp.max(St, axis=1, keepdims=True)
        matched = St >= m
        idxf = jnp.min(jnp.where(matched, It, jnp.float32(1e9)),
                       axis=1, keepdims=True)
        vals.append(m)
        idxs.append(idxf)
        if t < K - 1:
            St = jnp.where(matched, NEG, St)

    cv_ref[:, :8] = jnp.concatenate(
        vals + [jnp.full((B, 3), NEG, jnp.float32)], axis=1)
    ixs8 = jnp.concatenate(idxs + [jnp.zeros((B, 3), jnp.float32)], axis=1)
    ci_ref[:, :8] = ixs8

    @pl.when(j == pl.num_programs(0) - 1)
    def _fin():
        out_ref[...] = ixs8.astype(jnp.int32)


def _run_topk(cf, tf, spk2, tgt2):
    return pl.pallas_call(
        _topk_body,
        grid=(NSTEP,),
        in_specs=[
            pl.BlockSpec((B, D), lambda j: (0, 0)),
            pl.BlockSpec((NB, D), lambda j: (j, 0)),
            pl.BlockSpec((1, NB), lambda j: (0, j)),
            pl.BlockSpec((B, 1), lambda j: (0, 0)),
        ],
        out_specs=pl.BlockSpec((B, 8), lambda j: (0, 0)),
        out_shape=jax.ShapeDtypeStruct((B, 8), jnp.int32),
        scratch_shapes=[
            pltpu.VMEM((B, D), jnp.bfloat16),
            pltpu.VMEM((B, CW), jnp.float32),
            pltpu.VMEM((B, CW), jnp.float32),
            pltpu.VMEM((B, NB), jnp.float32),
        ],
        compiler_params=pltpu.CompilerParams(
            dimension_semantics=("arbitrary",)),
    )(cf, tf, spk2, tgt2)


_NC = 2
_NS = 16
_NW = _NC * _NS
_QPW = B // _NW
_RPW = _QPW * 8


def _gather_mean_body(idx_hbm, tab_hbm, out_hbm, idx_v, rows_v, out_v, sem):
    wid = lax.axis_index("s") * _NC + lax.axis_index("c")
    fbase = wid * _RPW
    qbase = wid * _QPW
    pltpu.sync_copy(idx_hbm.at[pl.ds(fbase, _RPW)], idx_v)
    pltpu.async_copy(tab_hbm.at[idx_v], rows_v, sem).wait()

    def qloop(q, _):
        def cloop(c, _):
            col = c * 16
            acc = rows_v[q * 8, pl.ds(col, 16)]
            for k in range(1, K):
                acc = acc + rows_v[q * 8 + k, pl.ds(col, 16)]
            out_v[q, pl.ds(col, 16)] = acc * (1.0 / K)
            return 0
        lax.fori_loop(0, D // 16, cloop, 0)
        return 0
    lax.fori_loop(0, _QPW, qloop, 0)
    pltpu.sync_copy(out_v, out_hbm.at[pl.ds(qbase, _QPW)])


def _run_gather_mean(idx_flat, tf):
    fn = functools.partial(
        pl.kernel,
        mesh=plsc.VectorSubcoreMesh(core_axis_name="c", subcore_axis_name="s"),
        out_type=jax.ShapeDtypeStruct((B, D), jnp.float32),
        scratch_types=[
            pltpu.VMEM((_RPW,), jnp.int32),
            pltpu.VMEM((_RPW, D), jnp.float32),
            pltpu.VMEM((_QPW, D), jnp.float32),
            pltpu.SemaphoreType.DMA,
        ],
    )(_gather_mean_body)
    return fn(idx_flat, tf)


def _mlp_body(cf_ref, rm_ref, w1_ref, b1_ref, w2_ref, b2_ref, out_ref,
              comb_ref):
    comb_ref[:, :D] = cf_ref[...]
    comb_ref[:, D:] = rm_ref[...]
    h = lax.dot_general(comb_ref[...], w1_ref[...], _DN,
                        preferred_element_type=jnp.float32) + b1_ref[...]
    h = h * jax.nn.sigmoid(h)
    out_ref[...] = lax.dot_general(h, w2_ref[...], _DN,
                                   preferred_element_type=jnp.float32) \
        + b2_ref[...]


def _run_mlp(cf, rm, w1, b1, w2, b2):
    return pl.pallas_call(
        _mlp_body,
        out_shape=jax.ShapeDtypeStruct((B, D), jnp.float32),
        scratch_shapes=[pltpu.VMEM((B, 2 * D), jnp.float32)],
    )(cf, rm, w1, b1, w2, b2)


def kernel(content_features, target_speaker_id, training_features,
           speaker_ids, W1, b1, W2, b2):
    cf = content_features.astype(jnp.float32)
    tf = training_features.astype(jnp.float32)
    spk2 = speaker_ids.astype(jnp.int32).reshape(1, N)
    tgt2 = target_speaker_id.astype(jnp.int32).reshape(B, 1)

    top8 = _run_topk(cf, tf, spk2, tgt2)
    idx_flat = top8.reshape(-1)
    rm = _run_gather_mean(idx_flat, tf)

    return _run_mlp(cf, rm, W1, b1.reshape(1, D), W2, b2.reshape(1, D))


# speedup vs baseline: 1.3139x; 1.2991x over previous
import functools

import jax
import jax.numpy as jnp
from jax import lax
from jax.experimental import pallas as pl
from jax.experimental.pallas import tpu as pltpu
from jax.experimental.pallas import tpu_sc as plsc

B = 512
N = 50000
D = 768
K = 5
NB = 2048
NSTEP = (N + NB - 1) // NB
NEG = float("-inf")
_DN = (((1,), (1,)), ((), ()))


CW = 128


def _topk_body(cf_ref, tf_ref, spk_ref, tgt_ref, out_ref,
               qn_ref, cv_ref, ci_ref, io_ref):
    j = pl.program_id(0)

    @pl.when(j == 0)
    def _init():
        cf = cf_ref[...]
        nrm = jnp.sqrt(jnp.sum(cf * cf, axis=1, keepdims=True))
        qn_ref[...] = (cf / jnp.maximum(nrm, 1e-8)).astype(jnp.bfloat16)
        cv_ref[...] = jnp.full((B, CW), NEG, jnp.float32)
        ci_ref[...] = jnp.zeros((B, CW), jnp.float32)
        io_ref[...] = lax.broadcasted_iota(
            jnp.int32, (B, NB), 1).astype(jnp.float32)

    tf = tf_ref[...]
    nrm = jnp.sqrt(jnp.sum(tf * tf, axis=1, keepdims=True))
    cn = (tf / jnp.maximum(nrm, 1e-8)).astype(jnp.bfloat16)
    S = lax.dot_general(qn_ref[...], cn, _DN,
                        preferred_element_type=jnp.float32)
    colid = lax.broadcasted_iota(jnp.int32, (1, NB), 1)
    spk = jnp.where(colid < N - j * NB, spk_ref[...], -1)
    valid = spk == tgt_ref[...]

    offf = (j * NB).astype(jnp.float32)
    St = jnp.concatenate([cv_ref[...], jnp.where(valid, S, NEG)], axis=1)
    It = jnp.concatenate([ci_ref[...], io_ref[...] + offf], axis=1)
    vals, idxs = [], []
    for t in range(K):
        m = jnp.max(St, axis=1, keepdims=True)
        matched = St >= m
        idxf = jnp.min(jnp.where(matched, It, jnp.float32(1e9)),
                       axis=1, keepdims=True)
        vals.append(m)
        idxs.append(idxf)
        if t < K - 1:
            St = jnp.where(matched, NEG, St)

    cv_ref[:, :8] = jnp.concatenate(
        vals + [jnp.full((B, 3), NEG, jnp.float32)], axis=1)
    ixs8 = jnp.concatenate(idxs + [jnp.zeros((B, 3), jnp.float32)], axis=1)
    ci_ref[:, :8] = ixs8

    @pl.when(j == pl.num_programs(0) - 1)
    def _fin():
        out_ref[...] = ixs8.astype(jnp.int32)


def _run_topk(cf, tf, spk2, tgt2):
    return pl.pallas_call(
        _topk_body,
        grid=(NSTEP,),
        in_specs=[
            pl.BlockSpec((B, D), lambda j: (0, 0)),
            pl.BlockSpec((NB, D), lambda j: (j, 0)),
            pl.BlockSpec((1, NB), lambda j: (0, j)),
            pl.BlockSpec((B, 1), lambda j: (0, 0)),
        ],
        out_specs=pl.BlockSpec((B, 8), lambda j: (0, 0)),
        out_shape=jax.ShapeDtypeStruct((B, 8), jnp.int32),
        scratch_shapes=[
            pltpu.VMEM((B, D), jnp.bfloat16),
            pltpu.VMEM((B, CW), jnp.float32),
            pltpu.VMEM((B, CW), jnp.float32),
            pltpu.VMEM((B, NB), jnp.float32),
        ],
        compiler_params=pltpu.CompilerParams(
            dimension_semantics=("arbitrary",)),
    )(cf, tf, spk2, tgt2)


_NC = 2
_NS = 16
_NW = _NC * _NS
_QPW = B // _NW
_RPW = _QPW * K


def _gather_mean_body(idx_hbm, tab_hbm, out_hbm, idx_v, rows_v, out_v, sem):
    wid = lax.axis_index("s") * _NC + lax.axis_index("c")
    fbase = wid * _RPW
    qbase = wid * _QPW
    pltpu.sync_copy(idx_hbm.at[pl.ds(fbase, _RPW)], idx_v)
    pltpu.async_copy(tab_hbm.at[idx_v], rows_v, sem).wait()

    def qloop(q, _):
        def cloop(c, _):
            col = c * 16
            acc = rows_v[q * K, pl.ds(col, 16)]
            for k in range(1, K):
                acc = acc + rows_v[q * K + k, pl.ds(col, 16)]
            out_v[q, pl.ds(col, 16)] = acc * (1.0 / K)
            return 0
        lax.fori_loop(0, D // 16, cloop, 0)
        return 0
    lax.fori_loop(0, _QPW, qloop, 0)
    pltpu.sync_copy(out_v, out_hbm.at[pl.ds(qbase, _QPW)])


def _run_gather_mean(idx_flat, tf):
    fn = functools.partial(
        pl.kernel,
        mesh=plsc.VectorSubcoreMesh(core_axis_name="c", subcore_axis_name="s"),
        out_type=jax.ShapeDtypeStruct((B, D), jnp.float32),
        scratch_types=[
            pltpu.VMEM((_RPW,), jnp.int32),
            pltpu.VMEM((_RPW, D), jnp.float32),
            pltpu.VMEM((_QPW, D), jnp.float32),
            pltpu.SemaphoreType.DMA,
        ],
    )(_gather_mean_body)
    return fn(idx_flat, tf)


def _mlp_body(cf_ref, rm_ref, w1_ref, b1_ref, w2_ref, b2_ref, out_ref,
              comb_ref):
    comb_ref[:, :D] = cf_ref[...]
    comb_ref[:, D:] = rm_ref[...]
    h = lax.dot_general(comb_ref[...], w1_ref[...], _DN,
                        preferred_element_type=jnp.float32) + b1_ref[...]
    h = h * jax.nn.sigmoid(h)
    out_ref[...] = lax.dot_general(h, w2_ref[...], _DN,
                                   preferred_element_type=jnp.float32) \
        + b2_ref[...]


def _run_mlp(cf, rm, w1, b1, w2, b2):
    return pl.pallas_call(
        _mlp_body,
        out_shape=jax.ShapeDtypeStruct((B, D), jnp.float32),
        scratch_shapes=[pltpu.VMEM((B, 2 * D), jnp.float32)],
    )(cf, rm, w1, b1, w2, b2)


def kernel(content_features, target_speaker_id, training_features,
           speaker_ids, W1, b1, W2, b2):
    cf = content_features.astype(jnp.float32)
    tf = training_features.astype(jnp.float32)
    spk2 = speaker_ids.astype(jnp.int32).reshape(1, N)
    tgt2 = target_speaker_id.astype(jnp.int32).reshape(B, 1)

    top8 = _run_topk(cf, tf, spk2, tgt2)
    idx_flat = top8[:, :K].reshape(-1)
    rm = _run_gather_mean(idx_flat, tf)

    return _run_mlp(cf, rm, W1, b1.reshape(1, D), W2, b2.reshape(1, D))


# speedup vs baseline: 1.3400x; 1.0199x over previous
import dataclasses
import functools

import jax
import jax.numpy as jnp
from jax import lax
from jax.experimental import pallas as pl
from jax.experimental.pallas import tpu as pltpu
from jax.experimental.pallas import tpu_sc as plsc

B = 512
N = 50000
D = 768
K = 5
NB = 2048
NSTEP = (N + NB - 1) // NB
NEG = float("-inf")
_DN = (((1,), (1,)), ((), ()))


CW = 128


def _topk_body(cf_ref, tf_ref, spk_ref, tgt_ref, out_ref,
               qn_ref, cv_ref, ci_ref, io_ref):
    j = pl.program_id(0)

    @pl.when(j == 0)
    def _init():
        cf = cf_ref[...]
        nrm = jnp.sqrt(jnp.sum(cf * cf, axis=1, keepdims=True))
        qn_ref[...] = (cf / jnp.maximum(nrm, 1e-8)).astype(jnp.bfloat16)
        cv_ref[...] = jnp.full((B, CW), NEG, jnp.float32)
        ci_ref[...] = jnp.zeros((B, CW), jnp.float32)
        io_ref[...] = lax.broadcasted_iota(
            jnp.int32, (B, NB), 1).astype(jnp.float32)

    tf = tf_ref[...]
    nrm = jnp.sqrt(jnp.sum(tf * tf, axis=1, keepdims=True))
    cn = (tf / jnp.maximum(nrm, 1e-8)).astype(jnp.bfloat16)
    S = lax.dot_general(qn_ref[...], cn, _DN,
                        preferred_element_type=jnp.float32)
    colid = lax.broadcasted_iota(jnp.int32, (1, NB), 1)
    spk = jnp.where(colid < N - j * NB, spk_ref[...], -1)
    valid = spk == tgt_ref[...]

    offf = (j * NB).astype(jnp.float32)
    St = jnp.concatenate([cv_ref[...], jnp.where(valid, S, NEG)], axis=1)
    It = jnp.concatenate([ci_ref[...], io_ref[...] + offf], axis=1)
    vals, idxs = [], []
    for t in range(K):
        m = jnp.max(St, axis=1, keepdims=True)
        matched = St >= m
        idxf = jnp.min(jnp.where(matched, It, jnp.float32(1e9)),
                       axis=1, keepdims=True)
        vals.append(m)
        idxs.append(idxf)
        if t < K - 1:
            St = jnp.where(matched, NEG, St)

    cv_ref[:, :8] = jnp.concatenate(
        vals + [jnp.full((B, 3), NEG, jnp.float32)], axis=1)
    ixs8 = jnp.concatenate(idxs + [jnp.zeros((B, 3), jnp.float32)], axis=1)
    ci_ref[:, :8] = ixs8

    @pl.when(j == pl.num_programs(0) - 1)
    def _fin():
        out_ref[...] = ixs8.astype(jnp.int32)


def _run_topk(cf, tf, spk2, tgt2):
    return pl.pallas_call(
        _topk_body,
        grid=(NSTEP,),
        in_specs=[
            pl.BlockSpec((B, D), lambda j: (0, 0)),
            pl.BlockSpec((NB, D), lambda j: (j, 0)),
            pl.BlockSpec((1, NB), lambda j: (0, j)),
            pl.BlockSpec((B, 1), lambda j: (0, 0)),
        ],
        out_specs=pl.BlockSpec((B, 8), lambda j: (0, 0)),
        out_shape=jax.ShapeDtypeStruct((B, 8), jnp.int32),
        scratch_shapes=[
            pltpu.VMEM((B, D), jnp.bfloat16),
            pltpu.VMEM((B, CW), jnp.float32),
            pltpu.VMEM((B, CW), jnp.float32),
            pltpu.VMEM((B, NB), jnp.float32),
        ],
        compiler_params=pltpu.CompilerParams(
            dimension_semantics=("arbitrary",)),
    )(cf, tf, spk2, tgt2)


_NC = 2
_NS = 16
_NW = _NC * _NS
_QPW = B // _NW
_RPW = _QPW * K


def _gather_mean_body(idx_hbm, tab_hbm, out_hbm, idx8_v, idx_v, rows_v,
                      out_v, sem):
    wid = lax.axis_index("s") * _NC + lax.axis_index("c")
    pltpu.sync_copy(idx_hbm.at[pl.ds(wid * _QPW * 8, _QPW * 8)], idx8_v)
    lane = lax.iota(jnp.int32, 16)
    for k in range(K):
        g = plsc.load_gather(idx8_v, [lane * 8 + k])
        plsc.store_scatter(idx_v, [lane * K + k], g)
    pltpu.async_copy(tab_hbm.at[idx_v], rows_v, sem).wait()

    nc = D // 16

    @plsc.parallel_loop(0, _QPW * nc, step=1, unroll=8)
    def _sum(i):
        q = i // nc
        col = (i % nc) * 16
        acc = rows_v[q * K, pl.ds(col, 16)]
        for k in range(1, K):
            acc = acc + rows_v[q * K + k, pl.ds(col, 16)]
        out_v[q, pl.ds(col, 16)] = acc

    pltpu.sync_copy(out_v, out_hbm.at[pl.ds(wid * _QPW, _QPW)])


def _sc_compiler_params():
    cp = pltpu.CompilerParams()
    if "needs_layout_passes" in pltpu.CompilerParams.__dataclass_fields__:
        cp = dataclasses.replace(cp, needs_layout_passes=False)
    return cp


def _run_gather_mean(idx_flat8, tf):
    fn = functools.partial(
        pl.kernel,
        mesh=plsc.VectorSubcoreMesh(core_axis_name="c", subcore_axis_name="s"),
        compiler_params=_sc_compiler_params(),
        out_type=jax.ShapeDtypeStruct((B, D), jnp.float32),
        scratch_types=[
            pltpu.VMEM((_QPW * 8,), jnp.int32),
            pltpu.VMEM((_RPW,), jnp.int32),
            pltpu.VMEM((_RPW, D), jnp.float32),
            pltpu.VMEM((_QPW, D), jnp.float32),
            pltpu.SemaphoreType.DMA,
        ],
    )(_gather_mean_body)
    return fn(idx_flat8, tf)


def _mlp_body(cf_ref, rm_ref, w1_ref, b1_ref, w2_ref, b2_ref, out_ref,
              comb_ref):
    comb_ref[:, :D] = cf_ref[...]
    comb_ref[:, D:] = rm_ref[...] * jnp.float32(1.0 / K)
    h = lax.dot_general(comb_ref[...], w1_ref[...], _DN,
                        preferred_element_type=jnp.float32) + b1_ref[...]
    h = h * jax.nn.sigmoid(h)
    out_ref[...] = lax.dot_general(h, w2_ref[...], _DN,
                                   preferred_element_type=jnp.float32) \
        + b2_ref[...]


def _run_mlp(cf, rm, w1, b1, w2, b2):
    return pl.pallas_call(
        _mlp_body,
        out_shape=jax.ShapeDtypeStruct((B, D), jnp.float32),
        scratch_shapes=[pltpu.VMEM((B, 2 * D), jnp.float32)],
    )(cf, rm, w1, b1, w2, b2)


def kernel(content_features, target_speaker_id, training_features,
           speaker_ids, W1, b1, W2, b2):
    cf = content_features.astype(jnp.float32)
    tf = training_features.astype(jnp.float32)
    spk2 = speaker_ids.astype(jnp.int32).reshape(1, N)
    tgt2 = target_speaker_id.astype(jnp.int32).reshape(B, 1)

    top8 = _run_topk(cf, tf, spk2, tgt2)
    idx_flat8 = top8.reshape(-1)
    rm = _run_gather_mean(idx_flat8, tf)

    return _run_mlp(cf, rm, W1, b1.reshape(1, D), W2, b2.reshape(1, D))
